# baseline (speedup 1.0000x reference)
import jax
import jax.numpy as jnp
import numpy as np
from jax.experimental import pallas as pl
from jax.experimental.pallas import tpu as pltpu

PRE_NMS_TOP_N = 6000
POST_NMS_TOP_N = 1000
NMS_THRESH = 0.7
IM_H = 800.0
IM_W = 1216.0
BBOX_XFORM_CLIP = float(np.log(1000.0 / 16.0))
NEG = -1e9
B = 256
NB = 24
PAD = B * NB


def _pair_supp(qx1, qy1, qx2, qy2, qa, qk, bx1, by1, bx2, by2, ba):
    xx1 = jnp.maximum(qx1[:, :, None], bx1[:, None, :])
    yy1 = jnp.maximum(qy1[:, :, None], by1[:, None, :])
    xx2 = jnp.minimum(qx2[:, :, None], bx2[:, None, :])
    yy2 = jnp.minimum(qy2[:, :, None], by2[:, None, :])
    inter = jnp.maximum(xx2 - xx1, 0.0) * jnp.maximum(yy2 - yy1, 0.0)
    iou = inter / (qa[:, :, None] + ba[:, None, :] - inter + 1e-9)
    hit = jnp.where(iou > NMS_THRESH, 1.0, 0.0) * qk[:, :, None]
    return jnp.max(hit, axis=1)


def _nms_body(ts_ref, a0r, a1r, a2r, a3r, r0r, r1r, r2r, r3r, out_ref,
              px1_s, py1_s, px2_s, py2_s, ar_s, kept_s, cnt_s):
    a0 = a0r[...]
    a1 = a1r[...]
    a2 = a2r[...]
    a3 = a3r[...]
    widths = a2 - a0 + 1.0
    heights = a3 - a1 + 1.0
    ctr_x = a0 + 0.5 * widths
    ctr_y = a1 + 0.5 * heights
    dw = jnp.minimum(r2r[...], BBOX_XFORM_CLIP)
    dh = jnp.minimum(r3r[...], BBOX_XFORM_CLIP)
    pred_ctr_x = r0r[...] * widths + ctr_x
    pred_ctr_y = r1r[...] * heights + ctr_y
    pred_w = jnp.exp(dw) * widths
    pred_h = jnp.exp(dh) * heights
    px1 = jnp.clip(pred_ctr_x - 0.5 * pred_w, 0.0, IM_W - 1.0)
    py1 = jnp.clip(pred_ctr_y - 0.5 * pred_h, 0.0, IM_H - 1.0)
    px2 = jnp.clip(pred_ctr_x + 0.5 * pred_w - 1.0, 0.0, IM_W - 1.0)
    py2 = jnp.clip(pred_ctr_y + 0.5 * pred_h - 1.0, 0.0, IM_H - 1.0)
    px1_s[...] = px1
    py1_s[...] = py1
    px2_s[...] = px2
    py2_s[...] = py2
    ar_s[...] = jnp.maximum(px2 - px1, 0.0) * jnp.maximum(py2 - py1, 0.0)

    n = a0.shape[0]
    kept_s[...] = jnp.zeros((n, NB, B), jnp.float32)
    cnt_s[...] = jnp.zeros((n, 128), jnp.float32)
    out_ref[...] = jnp.zeros((n, 1024, 128), jnp.float32)

    iota_i = jax.lax.broadcasted_iota(jnp.int32, (B, B), 1)
    iota_j = jax.lax.broadcasted_iota(jnp.int32, (B, B), 0)
    lt = jnp.where(iota_j < iota_i, 1.0, 0.0)
    iota_b = jax.lax.broadcasted_iota(jnp.int32, (n, B), 1).astype(jnp.float32)
    iota_p = jax.lax.broadcasted_iota(jnp.int32, (n, 1024, B), 1).astype(jnp.float32)
    ci = jax.lax.broadcasted_iota(jnp.int32, (n, B, 128), 2)

    def chunk2d(ref, c):
        return jnp.reshape(ref[:, pl.ds(c, 1), :], (n, B))

    for b in range(NB):
        @pl.when(jnp.min(cnt_s[:, 0:1]) < float(POST_NMS_TOP_N))
        def _process(b=b):
            bx1 = px1_s[:, b, :]
            by1 = py1_s[:, b, :]
            bx2 = px2_s[:, b, :]
            by2 = py2_s[:, b, :]
            ba = ar_s[:, b, :]
            bts = ts_ref[:, b, :]

            if b:
                def chunk(c, supp):
                    s = _pair_supp(chunk2d(px1_s, c), chunk2d(py1_s, c),
                                   chunk2d(px2_s, c), chunk2d(py2_s, c),
                                   chunk2d(ar_s, c), chunk2d(kept_s, c),
                                   bx1, by1, bx2, by2, ba)
                    return jnp.maximum(supp, s)
                supp = jax.lax.fori_loop(0, b, chunk, jnp.zeros((n, B), jnp.float32))
            else:
                supp = jnp.zeros((n, B), jnp.float32)

            gate = jnp.where((iota_b + float(b * B)) < float(PRE_NMS_TOP_N), 1.0, 0.0)
            pre = gate * (1.0 - supp)

            xx1 = jnp.maximum(bx1[:, :, None], bx1[:, None, :])
            yy1 = jnp.maximum(by1[:, :, None], by1[:, None, :])
            xx2 = jnp.minimum(bx2[:, :, None], bx2[:, None, :])
            yy2 = jnp.minimum(by2[:, :, None], by2[:, None, :])
            inter = jnp.maximum(xx2 - xx1, 0.0) * jnp.maximum(yy2 - yy1, 0.0)
            iou = inter / (ba[:, :, None] + ba[:, None, :] - inter + 1e-9)
            adj = jnp.where(iou > NMS_THRESH, 1.0, 0.0) * lt[None, :, :]

            def f(a):
                s = jnp.max(a[:, :, None] * adj, axis=1)
                return pre * (1.0 - s)

            prev = pre
            cur = f(pre)

            def w_cond(pc):
                return jnp.sum(jnp.abs(pc[0] - pc[1])) > 0.0

            def w_body(pc):
                return pc[1], f(pc[1])

            _, kept = jax.lax.while_loop(w_cond, w_body, (prev, cur))

            pos = jax.lax.dot_general(kept, lt, (((1,), (0,)), ((), ())),
                                      preferred_element_type=jnp.float32)
            gpos = cnt_s[:, 0:1] + pos
            w = jnp.where(iota_p == gpos[:, None, :], 1.0, 0.0) * kept[:, None, :]
            payload = (jnp.where(ci == 0, bx1[:, :, None], 0.0)
                       + jnp.where(ci == 1, by1[:, :, None], 0.0)
                       + jnp.where(ci == 2, bx2[:, :, None], 0.0)
                       + jnp.where(ci == 3, by2[:, :, None], 0.0)
                       + jnp.where(ci == 4, bts[:, :, None], 0.0))
            out_ref[...] += jax.lax.dot_general(
                w, payload, (((2,), (1,)), ((0,), (0,))),
                precision=jax.lax.Precision.HIGHEST,
                preferred_element_type=jnp.float32)
            cnt_s[:, 0:1] += jnp.sum(kept, axis=1, keepdims=True)
            kept_s[:, b, :] = kept

    cnt = cnt_s[:, 0:1]
    ci3 = jax.lax.broadcasted_iota(jnp.int32, (n, 1, 128), 2)
    fv = (jnp.where(ci3 == 0, px1_s[:, 0:1, 0:1], 0.0)
          + jnp.where(ci3 == 1, py1_s[:, 0:1, 0:1], 0.0)
          + jnp.where(ci3 == 2, px2_s[:, 0:1, 0:1], 0.0)
          + jnp.where(ci3 == 3, py2_s[:, 0:1, 0:1], 0.0)
          + jnp.where(ci3 == 4, ts_ref[:, 0:1, 0:1], 0.0))
    slot = jax.lax.broadcasted_iota(jnp.int32, (n, 1024, 1), 1).astype(jnp.float32)
    out_ref[...] = jnp.where(slot >= cnt[:, :, None], fv, out_ref[...])


def kernel(objectness, box_regression, anchors):
    N, A, H, W = objectness.shape
    obj = objectness.reshape(N, A, 1, H, W).transpose(0, 3, 4, 1, 2).reshape(N, -1)
    box_reg = box_regression.reshape(N, A, 4, H, W).transpose(0, 3, 4, 1, 2).reshape(N, -1, 4)
    scores_all = jax.nn.sigmoid(obj)
    top_scores, topk_idx = jax.lax.top_k(scores_all, PRE_NMS_TOP_N)
    bidx = jnp.arange(N)[:, None]
    box_sel = box_reg[bidx, topk_idx]
    anc_sel = anchors[bidx, topk_idx]

    pad = PAD - PRE_NMS_TOP_N
    ts = jnp.pad(top_scores, ((0, 0), (0, pad)),
                 constant_values=NEG).reshape(N, NB, B)
    cols = []
    for src in (anc_sel, box_sel):
        for c in range(4):
            cols.append(jnp.pad(src[:, :, c], ((0, 0), (0, pad))).reshape(N, NB, B))

    out = pl.pallas_call(
        _nms_body,
        out_shape=jax.ShapeDtypeStruct((N, 1024, 128), jnp.float32),
        scratch_shapes=[pltpu.VMEM((N, NB, B), jnp.float32)] * 6
        + [pltpu.VMEM((N, 128), jnp.float32)],
    )(ts, *cols)
    return out[:, :POST_NMS_TOP_N, 0:4], out[:, :POST_NMS_TOP_N, 4]
